# baseline (speedup 1.0000x reference)
import math

import jax
import jax.numpy as jnp
from jax.experimental import pallas as pl

_N = 10000
_NH = 64
_DEPTH = 3
_RATIO = 0.5
_RADIUS = 2.0
_KMAX = 32
_KNN = 3


def _mlp_apply(layers, x, last_act=True):
    n = len(layers)
    for i, l in enumerate(layers):
        x = x @ l["W"] + l["b"]
        if i < n - 1 or last_act:
            x = jax.nn.relu(x)
    return x


def _fps(pos, n_sample):
    p = jax.lax.stop_gradient(pos)

    def body(i, state):
        dists, idxs = state
        d = jnp.sum((p - p[idxs[i - 1]]) ** 2, axis=-1)
        dists = jnp.minimum(dists, d)
        return (dists, idxs.at[i].set(jnp.argmax(dists).astype(jnp.int32)))

    dists0 = jnp.full((p.shape[0],), jnp.inf, jnp.float32)
    idxs0 = jnp.zeros((n_sample,), jnp.int32)
    _, idxs = jax.lax.fori_loop(1, n_sample, body, (dists0, idxs0))
    return idxs


def _knn(pos_src, pos_dst, k):
    ps = jax.lax.stop_gradient(pos_src)
    pd = jax.lax.stop_gradient(pos_dst)
    d2 = jnp.sum(pd ** 2, axis=1)[:, None] + jnp.sum(ps ** 2, axis=1)[None, :] - 2.0 * (pd @ ps.T)
    d2 = jnp.maximum(d2, 0.0)
    negv, idx = jax.lax.top_k(-d2, k)
    return idx, -negv


def _point_conv(x, pos, center_idx, layers):
    centers = pos[center_idx]
    nbr, d2 = _knn(pos, centers, _KMAX)
    mask = d2 <= _RADIUS * _RADIUS
    xj = x[nbr]
    pj = pos[nbr] - centers[:, None, :]
    h = _mlp_apply(layers, jnp.concatenate([xj, pj], axis=-1), last_act=True)
    h = jnp.where(mask[..., None], h, -jnp.inf)
    out = jnp.max(h, axis=1)
    return jnp.where(jnp.isfinite(out), out, 0.0)


def _knn_interpolate(x, pos_src, pos_dst, k):
    idx, _ = _knn(pos_src, pos_dst, k)
    d2 = jnp.sum((pos_dst[:, None, :] - pos_src[idx]) ** 2, axis=-1)
    w = 1.0 / jnp.maximum(d2, 1e-16)
    w = w / jnp.sum(w, axis=1, keepdims=True)
    return jnp.sum(x[idx] * w[..., None], axis=1)


def _id_body(x_ref, o_ref):
    o_ref[...] = x_ref[...]


def kernel(x, pos, batch, norm, params):
    h = _mlp_apply(params["lin_in"], x, last_act=True)
    sa = [(h, pos)]
    cur_h, cur_pos = h, pos
    for i in range(_DEPTH):
        n_s = int(math.ceil(_RATIO * cur_pos.shape[0]))
        idx = _fps(cur_pos, n_s)
        cur_h = _point_conv(cur_h, cur_pos, idx, params["sa"][i])
        cur_pos = cur_pos[idx]
        sa.append((cur_h, cur_pos))
    fh, fpos = sa[-1]
    for i in range(_DEPTH):
        skip_h, skip_pos = sa[_DEPTH - 1 - i]
        interp = _knn_interpolate(fh, fpos, skip_pos, _KNN)
        fh = _mlp_apply(params["fp"][i], jnp.concatenate([interp, skip_h], axis=1), last_act=True)
        fpos = skip_pos
    out = _mlp_apply(params["lin_out"], fh, last_act=False)
    return pl.pallas_call(
        _id_body,
        out_shape=jax.ShapeDtypeStruct(out.shape, out.dtype),
    )(out)


# speedup vs baseline: 7.3737x; 7.3737x over previous
import functools
import math

import jax
import jax.numpy as jnp
from jax import lax
from jax.experimental import pallas as pl
from jax.experimental.pallas import tpu as pltpu
from jax.experimental.pallas import tpu_sc as plsc

_N = 10000
_NH = 64
_DEPTH = 3
_KMAX = 32
_KNN = 3
_R2 = 4.0
_BIG_I = 2147483647
_F32 = jnp.float32


def _pad128(n):
    return ((n + 127) // 128) * 128


def _mlp2_body(x_ref, w1_ref, b1_ref, w2_ref, b2_ref, o_ref, *, last_act):
    h = jnp.dot(x_ref[...], w1_ref[...], preferred_element_type=_F32) + b1_ref[...]
    h = jnp.maximum(h, 0.0)
    o = jnp.dot(h, w2_ref[...], preferred_element_type=_F32) + b2_ref[...]
    if last_act:
        o = jnp.maximum(o, 0.0)
    o_ref[...] = o


def _mlp2(x, w1, b1, w2, b2, last_act):
    n = x.shape[0]
    dout = w2.shape[1]
    return pl.pallas_call(
        functools.partial(_mlp2_body, last_act=last_act),
        out_shape=jax.ShapeDtypeStruct((n, dout), _F32),
    )(x, w1, b1.reshape(1, -1), w2, b2.reshape(1, -1))


def _fps_body(px_ref, py_ref, pz_ref, o_ref, dists_ref, *, n_valid, n_s):
    rows = px_ref.shape[0]
    flat = (lax.broadcasted_iota(jnp.int32, (rows, 128), 0) * 128
            + lax.broadcasted_iota(jnp.int32, (rows, 128), 1))
    valid = flat < n_valid
    px = px_ref[...]
    py = py_ref[...]
    pz = pz_ref[...]
    o_ref[...] = jnp.zeros_like(o_ref)
    dists_ref[...] = jnp.where(valid, jnp.inf, -jnp.inf).astype(_F32)

    first = flat == 0
    cx0 = jnp.sum(jnp.where(first, px, 0.0))
    cy0 = jnp.sum(jnp.where(first, py, 0.0))
    cz0 = jnp.sum(jnp.where(first, pz, 0.0))
    o_ref[0:1, :] = jnp.concatenate(
        [cx0.reshape(1, 1), cy0.reshape(1, 1), cz0.reshape(1, 1)], axis=1)

    def body(i, carry):
        cx, cy, cz = carry
        d = (px - cx) ** 2 + (py - cy) ** 2 + (pz - cz) ** 2
        dd = jnp.minimum(dists_ref[...], d)
        dists_ref[...] = dd
        m = jnp.max(dd)
        sel = jnp.min(jnp.where(dd == m, flat, _BIG_I))
        pick = flat == sel
        ncx = jnp.sum(jnp.where(pick, px, 0.0))
        ncy = jnp.sum(jnp.where(pick, py, 0.0))
        ncz = jnp.sum(jnp.where(pick, pz, 0.0))
        o_ref[pl.ds(i, 1), :] = jnp.concatenate(
            [ncx.reshape(1, 1), ncy.reshape(1, 1), ncz.reshape(1, 1)], axis=1)
        return (ncx, ncy, ncz)

    lax.fori_loop(1, n_s, body, (cx0, cy0, cz0))


def _fps(pos_mat, n_valid, n_s):
    pn = pos_mat.shape[0]
    rows = pn // 128
    cols = [pos_mat[:, i].reshape(rows, 128) for i in range(3)]
    q = _pad128(n_s)
    return pl.pallas_call(
        functools.partial(_fps_body, n_valid=n_valid, n_s=n_s),
        out_shape=jax.ShapeDtypeStruct((q, 3), _F32),
        scratch_shapes=[pltpu.VMEM((rows, 128), _F32)],
    )(*cols)


def _knn_body(q3t_ref, s38_ref, oi_ref, od_ref, dmat_ref, *, n_src, k):
    spad = s38_ref.shape[0]
    q3t = q3t_ref[...]
    s38 = s38_ref[...]
    qn = (q3t[0:1, :] * q3t[0:1, :] + q3t[1:2, :] * q3t[1:2, :]) \
        + q3t[2:3, :] * q3t[2:3, :]
    sn = (s38[:, 0:1] * s38[:, 0:1] + s38[:, 1:2] * s38[:, 1:2]) \
        + s38[:, 2:3] * s38[:, 2:3]
    mm = jnp.dot(s38, q3t, preferred_element_type=_F32)
    d = jnp.maximum((qn + sn) - 2.0 * mm, 0.0)
    srow = lax.broadcasted_iota(jnp.int32, (spad, 128), 0)
    d = jnp.where(srow < n_src, d, jnp.inf)
    dmat_ref[...] = d

    def ext(j, _):
        dv = dmat_ref[...]
        m = jnp.min(dv, axis=0, keepdims=True)
        sel = jnp.min(jnp.where(dv == m, srow, _BIG_I), axis=0, keepdims=True)
        od_ref[pl.ds(j, 1), :] = m
        oi_ref[pl.ds(j, 1), :] = sel
        dmat_ref[...] = jnp.where(srow == sel, jnp.inf, dv)
        return 0

    lax.fori_loop(0, k, ext, 0)


def _knn(q_mat, n_q_pad, src_mat, n_src, k):
    qpad = q_mat.shape[0]
    spad = src_mat.shape[0]
    q3t = jnp.pad(q_mat, ((0, 0), (0, 5))).T
    s38 = jnp.pad(src_mat, ((0, 0), (0, 5)))
    grid = (qpad // 128,)
    ospec = pl.BlockSpec((k, 128), lambda b: (0, b))
    return pl.pallas_call(
        functools.partial(_knn_body, n_src=n_src, k=k),
        grid=grid,
        in_specs=[pl.BlockSpec((8, 128), lambda b: (0, b)),
                  pl.BlockSpec((spad, 8), lambda b: (0, 0))],
        out_specs=[ospec, ospec],
        out_shape=[jax.ShapeDtypeStruct((k, qpad), jnp.int32),
                   jax.ShapeDtypeStruct((k, qpad), _F32)],
        scratch_shapes=[pltpu.VMEM((spad, 128), _F32)],
    )(q3t, s38)


def _sc_gather(table, idx):
    b_total = idx.shape[0]
    d = table.shape[1]
    info = plsc.get_sparse_core_info()
    nw = info.num_cores * info.num_subcores
    b_per_w = b_total // nw
    max_rows = 110000 // d
    chunk = min(b_per_w, max_rows) & ~7
    while b_per_w % chunk:
        chunk -= 8
    nchunks = b_per_w // chunk
    mesh = plsc.VectorSubcoreMesh(core_axis_name="c", subcore_axis_name="s")

    @functools.partial(
        pl.kernel, mesh=mesh,
        out_type=jax.ShapeDtypeStruct((b_total, d), _F32),
        scratch_types=[
            pltpu.VMEM((chunk,), jnp.int32),
            pltpu.VMEM((chunk, d), _F32),
            pltpu.SemaphoreType.DMA,
        ],
    )
    def gather_k(table_hbm, idx_hbm, out_hbm, idx_v, rows_v, sem):
        wid = lax.axis_index("s") * info.num_cores + lax.axis_index("c")
        base = wid * b_per_w
        for c in range(nchunks):
            off = base + c * chunk
            pltpu.sync_copy(idx_hbm.at[pl.ds(off, chunk)], idx_v)
            pltpu.async_copy(table_hbm.at[idx_v], rows_v, sem).wait()
            pltpu.sync_copy(rows_v, out_hbm.at[pl.ds(off, chunk)])

    return gather_k(table, idx)


def _conv_body(t_ref, cpad_ref, d2t_ref, w1_ref, b1_ref, w2_ref, b2_ref, o_ref):
    bq = o_ref.shape[0]
    cpad = cpad_ref[...]
    b1 = b1_ref[...]
    b2 = b2_ref[...]
    acc = jnp.full((bq, _NH), -jnp.inf, _F32)
    for j in range(_KMAX):
        t2 = t_ref[j] - cpad
        h1 = jnp.maximum(jnp.dot(t2, w1_ref[...], preferred_element_type=_F32) + b1, 0.0)
        h2 = jnp.maximum(jnp.dot(h1, w2_ref[...], preferred_element_type=_F32) + b2, 0.0)
        mj = d2t_ref[:, pl.ds(j, 1)] <= _R2
        acc = jnp.maximum(acc, jnp.where(mj, h2, -jnp.inf))
    o_ref[...] = jnp.where(acc > -3e38, acc, 0.0)


def _conv(t_g, cpad, d2t, w1full, b1p, w2p, b2):
    qpad = cpad.shape[0]
    bq = 256
    grid = (qpad // bq,)
    return pl.pallas_call(
        _conv_body,
        grid=grid,
        in_specs=[
            pl.BlockSpec((_KMAX, bq, 128), lambda b: (0, b, 0)),
            pl.BlockSpec((bq, 128), lambda b: (b, 0)),
            pl.BlockSpec((bq, _KMAX), lambda b: (b, 0)),
            pl.BlockSpec((128, 128), lambda b: (0, 0)),
            pl.BlockSpec((1, 128), lambda b: (0, 0)),
            pl.BlockSpec((128, _NH), lambda b: (0, 0)),
            pl.BlockSpec((1, _NH), lambda b: (0, 0)),
        ],
        out_specs=pl.BlockSpec((bq, _NH), lambda b: (b, 0)),
        out_shape=jax.ShapeDtypeStruct((qpad, _NH), _F32),
    )(t_g, cpad, d2t, w1full, b1p, w2p, b2)


def _fp_body(g_ref, dst8_ref, skip_ref, w1_ref, b1_ref, w2_ref,
             b2_ref, o_ref):
    dst = dst8_ref[...]
    ws = []
    for kk in range(_KNN):
        gk = g_ref[kk]
        dx = dst[:, 0:1] - gk[:, 64:65]
        dy = dst[:, 1:2] - gk[:, 65:66]
        dz = dst[:, 2:3] - gk[:, 66:67]
        d2 = (dx * dx + dy * dy) + dz * dz
        ws.append(1.0 / jnp.maximum(d2, 1e-16))
    tot = (ws[0] + ws[1]) + ws[2]
    interp = ((g_ref[0][:, :_NH] * (ws[0] / tot)
               + g_ref[1][:, :_NH] * (ws[1] / tot))
              + g_ref[2][:, :_NH] * (ws[2] / tot))
    cc = jnp.concatenate([interp, skip_ref[...]], axis=1)
    h1 = jnp.maximum(jnp.dot(cc, w1_ref[...], preferred_element_type=_F32)
                     + b1_ref[...], 0.0)
    o = jnp.dot(h1, w2_ref[...], preferred_element_type=_F32) + b2_ref[...]
    o_ref[...] = jnp.maximum(o, 0.0)


def _fp(g, dst8, skip_h, w1, b1, w2, b2):
    qpad = skip_h.shape[0]
    bq = 512
    grid = (qpad // bq,)
    return pl.pallas_call(
        _fp_body,
        grid=grid,
        in_specs=[
            pl.BlockSpec((_KNN, bq, 128), lambda b: (0, b, 0)),
            pl.BlockSpec((bq, 8), lambda b: (b, 0)),
            pl.BlockSpec((bq, _NH), lambda b: (b, 0)),
            pl.BlockSpec((2 * _NH, 2 * _NH), lambda b: (0, 0)),
            pl.BlockSpec((1, 2 * _NH), lambda b: (0, 0)),
            pl.BlockSpec((2 * _NH, _NH), lambda b: (0, 0)),
            pl.BlockSpec((1, _NH), lambda b: (0, 0)),
        ],
        out_specs=pl.BlockSpec((bq, _NH), lambda b: (b, 0)),
        out_shape=jax.ShapeDtypeStruct((qpad, _NH), _F32),
    )(g, dst8, skip_h, w1, b1, w2, b2)


def kernel(x, pos, batch, norm, params):
    del batch, norm
    p0 = 10240
    x8 = jnp.pad(x, ((0, p0 - _N), (0, 5)))
    pos_p = jnp.pad(pos, ((0, p0 - _N), (0, 0)))

    li = params["lin_in"]
    w1 = jnp.pad(li[0]["W"], ((0, 5), (0, 0)))
    h = _mlp2(x8, w1, li[0]["b"], li[1]["W"], li[1]["b"], last_act=True)

    sa = [(h, pos_p, _N)]
    cur_h, cur_pos, n_cur = h, pos_p, _N
    for lvl in range(_DEPTH):
        n_s = int(math.ceil(0.5 * n_cur))
        qpad = _pad128(n_s)
        pn = cur_pos.shape[0]
        layers = params["sa"][lvl]
        w1f, b1f = layers[0]["W"], layers[0]["b"]
        w2f, b2f = layers[1]["W"], layers[1]["b"]
        w1full = jnp.pad(w1f, ((0, 61), (0, 61)))
        b1p = jnp.pad(b1f, (0, 61)).reshape(1, 128)
        w2p = jnp.pad(w2f, ((0, 61), (0, 0)))

        centers = _fps(cur_pos, n_cur, n_s)
        nbr, d2 = _knn(centers, qpad, cur_pos, n_cur, _KMAX)
        tbl = jnp.concatenate(
            [cur_h, cur_pos, jnp.zeros((pn, 61), _F32)], axis=1)
        t_g = _sc_gather(tbl, nbr.reshape(-1)).reshape(_KMAX, qpad, 128)
        cpad = jnp.pad(centers, ((0, 0), (_NH, 61)))
        d2t = d2.T
        cur_h = _conv(t_g, cpad, d2t, w1full, b1p, w2p, b2f.reshape(1, _NH))
        cur_pos = centers
        n_cur = n_s
        sa.append((cur_h, cur_pos, n_cur))

    fh, fpos, n_f = sa[-1]
    for i in range(_DEPTH):
        skip_h, skip_pos, n_d = sa[_DEPTH - 1 - i]
        dpad = skip_pos.shape[0]
        layers = params["fp"][i]
        w1f, b1f = layers[0]["W"], layers[0]["b"]
        w2f, b2f = layers[1]["W"], layers[1]["b"]

        idx, d2 = _knn(skip_pos, dpad, fpos, n_f, _KNN)
        fpad = fh.shape[0]
        tbl = jnp.concatenate(
            [fh, fpos, jnp.zeros((fpad, 61), _F32)], axis=1)
        g = _sc_gather(tbl, idx.reshape(-1)).reshape(_KNN, dpad, 128)
        dst8 = jnp.pad(skip_pos, ((0, 0), (0, 5)))
        fh = _fp(g, dst8, skip_h, w1f,
                 b1f.reshape(1, -1), w2f, b2f.reshape(1, -1))
        fpos, n_f = skip_pos, n_d

    lo = params["lin_out"]
    out = _mlp2(fh, lo[0]["W"], lo[0]["b"], lo[1]["W"], lo[1]["b"],
                last_act=False)
    return out[:_N]


# speedup vs baseline: 8.1448x; 1.1046x over previous
import functools
import math

import jax
import jax.numpy as jnp
from jax import lax
from jax.experimental import pallas as pl
from jax.experimental.pallas import tpu as pltpu
from jax.experimental.pallas import tpu_sc as plsc

_N = 10000
_NH = 64
_DEPTH = 3
_KMAX = 32
_KNN = 3
_R2 = 4.0
_BIG_I = 2147483647
_F32 = jnp.float32


def _pad128(n):
    return ((n + 127) // 128) * 128


def _mlp2_body(x_ref, w1_ref, b1_ref, w2_ref, b2_ref, o_ref, *, last_act):
    h = jnp.dot(x_ref[...], w1_ref[...], preferred_element_type=_F32) + b1_ref[...]
    h = jnp.maximum(h, 0.0)
    o = jnp.dot(h, w2_ref[...], preferred_element_type=_F32) + b2_ref[...]
    if last_act:
        o = jnp.maximum(o, 0.0)
    o_ref[...] = o


def _mlp2(x, w1, b1, w2, b2, last_act):
    n = x.shape[0]
    dout = w2.shape[1]
    return pl.pallas_call(
        functools.partial(_mlp2_body, last_act=last_act),
        out_shape=jax.ShapeDtypeStruct((n, dout), _F32),
    )(x, w1, b1.reshape(1, -1), w2, b2.reshape(1, -1))


def _fps_body(px_ref, py_ref, pz_ref, o_ref, dists_ref, *, n_valid, n_s):
    rows = px_ref.shape[0]
    flat = (lax.broadcasted_iota(jnp.int32, (rows, 128), 0) * 128
            + lax.broadcasted_iota(jnp.int32, (rows, 128), 1))
    valid = flat < n_valid
    px = px_ref[...]
    py = py_ref[...]
    pz = pz_ref[...]
    o_ref[...] = jnp.zeros_like(o_ref)
    dists_ref[...] = jnp.where(valid, jnp.inf, -jnp.inf).astype(_F32)

    first = flat == 0
    cx0 = jnp.sum(jnp.where(first, px, 0.0))
    cy0 = jnp.sum(jnp.where(first, py, 0.0))
    cz0 = jnp.sum(jnp.where(first, pz, 0.0))
    o_ref[0:1, :] = jnp.concatenate(
        [cx0.reshape(1, 1), cy0.reshape(1, 1), cz0.reshape(1, 1)], axis=1)

    def body(i, carry):
        cx, cy, cz = carry
        d = (px - cx) ** 2 + (py - cy) ** 2 + (pz - cz) ** 2
        dd = jnp.minimum(dists_ref[...], d)
        dists_ref[...] = dd
        m = jnp.max(dd)
        sel = jnp.min(jnp.where(dd == m, flat, _BIG_I))
        pick = flat == sel
        ncx = jnp.sum(jnp.where(pick, px, 0.0))
        ncy = jnp.sum(jnp.where(pick, py, 0.0))
        ncz = jnp.sum(jnp.where(pick, pz, 0.0))
        o_ref[pl.ds(i, 1), :] = jnp.concatenate(
            [ncx.reshape(1, 1), ncy.reshape(1, 1), ncz.reshape(1, 1)], axis=1)
        return (ncx, ncy, ncz)

    lax.fori_loop(1, n_s, body, (cx0, cy0, cz0))


def _fps(pos_mat, n_valid, n_s):
    pn = pos_mat.shape[0]
    rows = pn // 128
    cols = [pos_mat[:, i].reshape(rows, 128) for i in range(3)]
    q = _pad128(n_s)
    return pl.pallas_call(
        functools.partial(_fps_body, n_valid=n_valid, n_s=n_s),
        out_shape=jax.ShapeDtypeStruct((q, 3), _F32),
        scratch_shapes=[pltpu.VMEM((rows, 128), _F32)],
    )(*cols)


def _knn_body(q3t_ref, s38_ref, oi_ref, od_ref, dmat_ref, *, n_src, k):
    spad = s38_ref.shape[0]
    q3t = q3t_ref[...]
    s38 = s38_ref[...]
    qn = (q3t[0:1, :] * q3t[0:1, :] + q3t[1:2, :] * q3t[1:2, :]) \
        + q3t[2:3, :] * q3t[2:3, :]
    sn = (s38[:, 0:1] * s38[:, 0:1] + s38[:, 1:2] * s38[:, 1:2]) \
        + s38[:, 2:3] * s38[:, 2:3]
    mm = jnp.dot(s38, q3t, preferred_element_type=_F32)
    d = jnp.maximum((qn + sn) - 2.0 * mm, 0.0)
    srow = lax.broadcasted_iota(jnp.int32, (spad, 128), 0)
    d = jnp.where(srow < n_src, d, jnp.inf)
    dmat_ref[...] = d

    def ext(j, carry):
        tv, tr = carry
        dv = dmat_ref[...]
        gt = (dv > tv) | ((dv == tv) & (srow > tr))
        cand = jnp.where(gt, dv, jnp.inf)
        m = jnp.min(cand, axis=0, keepdims=True)
        sel = jnp.min(jnp.where(cand == m, srow, _BIG_I), axis=0,
                      keepdims=True)
        od_ref[pl.ds(j, 1), :] = m
        oi_ref[pl.ds(j, 1), :] = sel
        return (m, sel)

    tv0 = jnp.full((1, 128), -1.0, _F32)
    tr0 = jnp.full((1, 128), -1, jnp.int32)
    lax.fori_loop(0, k, ext, (tv0, tr0))


def _knn(q_mat, n_q_pad, src_mat, n_src, k):
    qpad = q_mat.shape[0]
    spad = src_mat.shape[0]
    q3t = jnp.pad(q_mat, ((0, 0), (0, 5))).T
    s38 = jnp.pad(src_mat, ((0, 0), (0, 5)))
    grid = (qpad // 128,)
    ospec = pl.BlockSpec((k, 128), lambda b: (0, b))
    return pl.pallas_call(
        functools.partial(_knn_body, n_src=n_src, k=k),
        grid=grid,
        in_specs=[pl.BlockSpec((8, 128), lambda b: (0, b)),
                  pl.BlockSpec((spad, 8), lambda b: (0, 0))],
        out_specs=[ospec, ospec],
        out_shape=[jax.ShapeDtypeStruct((k, qpad), jnp.int32),
                   jax.ShapeDtypeStruct((k, qpad), _F32)],
        scratch_shapes=[pltpu.VMEM((spad, 128), _F32)],
    )(q3t, s38)


def _sc_gather(table, idx):
    b_total = idx.shape[0]
    d = table.shape[1]
    info = plsc.get_sparse_core_info()
    nw = info.num_cores * info.num_subcores
    b_per_w = b_total // nw
    max_rows = 110000 // d
    chunk = min(b_per_w, max_rows) & ~7
    while b_per_w % chunk:
        chunk -= 8
    nchunks = b_per_w // chunk
    mesh = plsc.VectorSubcoreMesh(core_axis_name="c", subcore_axis_name="s")

    @functools.partial(
        pl.kernel, mesh=mesh,
        out_type=jax.ShapeDtypeStruct((b_total, d), _F32),
        scratch_types=[
            pltpu.VMEM((chunk,), jnp.int32),
            pltpu.VMEM((chunk, d), _F32),
            pltpu.SemaphoreType.DMA,
        ],
    )
    def gather_k(table_hbm, idx_hbm, out_hbm, idx_v, rows_v, sem):
        wid = lax.axis_index("s") * info.num_cores + lax.axis_index("c")
        base = wid * b_per_w
        for c in range(nchunks):
            off = base + c * chunk
            pltpu.sync_copy(idx_hbm.at[pl.ds(off, chunk)], idx_v)
            pltpu.async_copy(table_hbm.at[idx_v], rows_v, sem).wait()
            pltpu.sync_copy(rows_v, out_hbm.at[pl.ds(off, chunk)])

    return gather_k(table, idx)


def _conv_body(t_ref, cpad_ref, d2t_ref, w1_ref, b1_ref, w2_ref, b2_ref, o_ref):
    bq = o_ref.shape[0]
    cpad = cpad_ref[...]
    b1 = b1_ref[...]
    b2 = b2_ref[...]
    acc = jnp.full((bq, _NH), -jnp.inf, _F32)
    for j in range(_KMAX):
        t2 = t_ref[j] - cpad
        h1 = jnp.maximum(jnp.dot(t2, w1_ref[...], preferred_element_type=_F32) + b1, 0.0)
        h2 = jnp.maximum(jnp.dot(h1, w2_ref[...], preferred_element_type=_F32) + b2, 0.0)
        mj = d2t_ref[:, pl.ds(j, 1)] <= _R2
        acc = jnp.maximum(acc, jnp.where(mj, h2, -jnp.inf))
    o_ref[...] = jnp.where(acc > -3e38, acc, 0.0)


def _conv(t_g, cpad, d2t, w1full, b1p, w2p, b2):
    qpad = cpad.shape[0]
    bq = 256
    grid = (qpad // bq,)
    return pl.pallas_call(
        _conv_body,
        grid=grid,
        in_specs=[
            pl.BlockSpec((_KMAX, bq, 128), lambda b: (0, b, 0)),
            pl.BlockSpec((bq, 128), lambda b: (b, 0)),
            pl.BlockSpec((bq, _KMAX), lambda b: (b, 0)),
            pl.BlockSpec((128, 128), lambda b: (0, 0)),
            pl.BlockSpec((1, 128), lambda b: (0, 0)),
            pl.BlockSpec((128, _NH), lambda b: (0, 0)),
            pl.BlockSpec((1, _NH), lambda b: (0, 0)),
        ],
        out_specs=pl.BlockSpec((bq, _NH), lambda b: (b, 0)),
        out_shape=jax.ShapeDtypeStruct((qpad, _NH), _F32),
    )(t_g, cpad, d2t, w1full, b1p, w2p, b2)


def _fp_body(g_ref, dst8_ref, skip_ref, w1_ref, b1_ref, w2_ref,
             b2_ref, o_ref):
    dst = dst8_ref[...]
    ws = []
    for kk in range(_KNN):
        gk = g_ref[kk]
        dx = dst[:, 0:1] - gk[:, 64:65]
        dy = dst[:, 1:2] - gk[:, 65:66]
        dz = dst[:, 2:3] - gk[:, 66:67]
        d2 = (dx * dx + dy * dy) + dz * dz
        ws.append(1.0 / jnp.maximum(d2, 1e-16))
    tot = (ws[0] + ws[1]) + ws[2]
    interp = ((g_ref[0][:, :_NH] * (ws[0] / tot)
               + g_ref[1][:, :_NH] * (ws[1] / tot))
              + g_ref[2][:, :_NH] * (ws[2] / tot))
    cc = jnp.concatenate([interp, skip_ref[...]], axis=1)
    h1 = jnp.maximum(jnp.dot(cc, w1_ref[...], preferred_element_type=_F32)
                     + b1_ref[...], 0.0)
    o = jnp.dot(h1, w2_ref[...], preferred_element_type=_F32) + b2_ref[...]
    o_ref[...] = jnp.maximum(o, 0.0)


def _fp(g, dst8, skip_h, w1, b1, w2, b2):
    qpad = skip_h.shape[0]
    bq = 512
    grid = (qpad // bq,)
    return pl.pallas_call(
        _fp_body,
        grid=grid,
        in_specs=[
            pl.BlockSpec((_KNN, bq, 128), lambda b: (0, b, 0)),
            pl.BlockSpec((bq, 8), lambda b: (b, 0)),
            pl.BlockSpec((bq, _NH), lambda b: (b, 0)),
            pl.BlockSpec((2 * _NH, 2 * _NH), lambda b: (0, 0)),
            pl.BlockSpec((1, 2 * _NH), lambda b: (0, 0)),
            pl.BlockSpec((2 * _NH, _NH), lambda b: (0, 0)),
            pl.BlockSpec((1, _NH), lambda b: (0, 0)),
        ],
        out_specs=pl.BlockSpec((bq, _NH), lambda b: (b, 0)),
        out_shape=jax.ShapeDtypeStruct((qpad, _NH), _F32),
    )(g, dst8, skip_h, w1, b1, w2, b2)


def kernel(x, pos, batch, norm, params):
    del batch, norm
    p0 = 10240
    x8 = jnp.pad(x, ((0, p0 - _N), (0, 5)))
    pos_p = jnp.pad(pos, ((0, p0 - _N), (0, 0)))

    li = params["lin_in"]
    w1 = jnp.pad(li[0]["W"], ((0, 5), (0, 0)))
    h = _mlp2(x8, w1, li[0]["b"], li[1]["W"], li[1]["b"], last_act=True)

    sa = [(h, pos_p, _N)]
    cur_h, cur_pos, n_cur = h, pos_p, _N
    for lvl in range(_DEPTH):
        n_s = int(math.ceil(0.5 * n_cur))
        qpad = _pad128(n_s)
        pn = cur_pos.shape[0]
        layers = params["sa"][lvl]
        w1f, b1f = layers[0]["W"], layers[0]["b"]
        w2f, b2f = layers[1]["W"], layers[1]["b"]
        w1full = jnp.pad(w1f, ((0, 61), (0, 61)))
        b1p = jnp.pad(b1f, (0, 61)).reshape(1, 128)
        w2p = jnp.pad(w2f, ((0, 61), (0, 0)))

        centers = _fps(cur_pos, n_cur, n_s)
        nbr, d2 = _knn(centers, qpad, cur_pos, n_cur, _KMAX)
        tbl = jnp.concatenate(
            [cur_h, cur_pos, jnp.zeros((pn, 61), _F32)], axis=1)
        t_g = _sc_gather(tbl, nbr.reshape(-1)).reshape(_KMAX, qpad, 128)
        cpad = jnp.pad(centers, ((0, 0), (_NH, 61)))
        d2t = d2.T
        cur_h = _conv(t_g, cpad, d2t, w1full, b1p, w2p, b2f.reshape(1, _NH))
        cur_pos = centers
        n_cur = n_s
        sa.append((cur_h, cur_pos, n_cur))

    fh, fpos, n_f = sa[-1]
    for i in range(_DEPTH):
        skip_h, skip_pos, n_d = sa[_DEPTH - 1 - i]
        dpad = skip_pos.shape[0]
        layers = params["fp"][i]
        w1f, b1f = layers[0]["W"], layers[0]["b"]
        w2f, b2f = layers[1]["W"], layers[1]["b"]

        idx, d2 = _knn(skip_pos, dpad, fpos, n_f, _KNN)
        fpad = fh.shape[0]
        tbl = jnp.concatenate(
            [fh, fpos, jnp.zeros((fpad, 61), _F32)], axis=1)
        g = _sc_gather(tbl, idx.reshape(-1)).reshape(_KNN, dpad, 128)
        dst8 = jnp.pad(skip_pos, ((0, 0), (0, 5)))
        fh = _fp(g, dst8, skip_h, w1f,
                 b1f.reshape(1, -1), w2f, b2f.reshape(1, -1))
        fpos, n_f = skip_pos, n_d

    lo = params["lin_out"]
    out = _mlp2(fh, lo[0]["W"], lo[0]["b"], lo[1]["W"], lo[1]["b"],
                last_act=False)
    return out[:_N]
